# Initial kernel scaffold; baseline (speedup 1.0000x reference)
import jax
import jax.numpy as jnp
from jax.experimental import pallas as pl


def kernel(node_feats, edge_feats, edge_index, mlpt_W, mlpt_b, mlpt_gamma, mlpt_beta, gate_W, gate_b, gate_gamma, gate_beta, node_gamma, node_beta):
    raise NotImplementedError("write your pallas kernel here")


# speedup vs baseline: 1.4007x; 1.4007x over previous
import functools

import jax
import jax.numpy as jnp
from jax import lax
from jax.experimental import pallas as pl
from jax.experimental.pallas import tpu as pltpu
from jax.experimental.pallas import tpu_sc as plsc

N = 10000
E = 320000
D = 128
F2 = 2 * D

NC = 2
NS = 16
NW = NC * NS
EPW = E // NW
B = 80
CH = EPW // B
NP = 10240
NPS = NP // NS

_mesh = plsc.VectorSubcoreMesh(core_axis_name="c", subcore_axis_name="s")


def _proj_body(nf_ref, ws_ref, wd_ref, ps_ref, pd_ref):
    x = nf_ref[...]
    ps_ref[...] = jnp.dot(x, ws_ref[...], preferred_element_type=jnp.float32)
    pd_ref[...] = jnp.dot(x, wd_ref[...], preferred_element_type=jnp.float32)


def _edge_mm_body(ef_ref, we_ref, c_ref):
    c_ref[...] = jnp.dot(ef_ref[...], we_ref[...],
                         preferred_element_type=jnp.float32)


def _sc_gather_body(ps_hbm, pd_hbm, src_hbm, dst_hbm, c_hbm,
                    lin_hbm, part_hbm,
                    si_v, di_v, a_v, b_v, c_v, acc_v, sem1, sem2, sem3):
    cid = lax.axis_index("c")
    sid = lax.axis_index("s")
    wid = sid * NC + cid
    base = wid * EPW

    zero = jnp.zeros((16,), jnp.float32)
    for j in range(2 * F2 // 16):
        acc_v[0, pl.ds(16 * j, 16)] = zero

    def chunk_body(t, _):
        off = base + t * B
        pltpu.sync_copy(src_hbm.at[pl.ds(off, B)], si_v)
        pltpu.sync_copy(dst_hbm.at[pl.ds(off, B)], di_v)
        cp1 = pltpu.async_copy(ps_hbm.at[si_v], a_v, sem1)
        cp2 = pltpu.async_copy(pd_hbm.at[di_v], b_v, sem2)
        cp3 = pltpu.async_copy(c_hbm.at[pl.ds(off, B)], c_v, sem3)
        cp1.wait()
        cp2.wait()
        cp3.wait()

        for j in range(F2 // 16):
            cs = pl.ds(16 * j, 16)

            def row_body(r, carry):
                s, q = carry
                v = a_v[r, cs] + b_v[r, cs] + c_v[r, cs]
                a_v[r, cs] = v
                return s + v, q + v * v

            s, q = lax.fori_loop(0, B, row_body, (zero, zero))
            acc_v[0, cs] = acc_v[0, cs] + s
            qs = pl.ds(F2 + 16 * j, 16)
            acc_v[0, qs] = acc_v[0, qs] + q

        pltpu.sync_copy(a_v, lin_hbm.at[pl.ds(off, B)])
        return 0

    lax.fori_loop(0, CH, chunk_body, 0)
    pltpu.sync_copy(acc_v, part_hbm.at[wid])


_sc_gather = functools.partial(
    pl.kernel,
    out_type=[jax.ShapeDtypeStruct((E, F2), jnp.float32),
              jax.ShapeDtypeStruct((NW, 1, 2 * F2), jnp.float32)],
    mesh=_mesh,
    scratch_types=[
        pltpu.VMEM((B,), jnp.int32),
        pltpu.VMEM((B,), jnp.int32),
        pltpu.VMEM((B, F2), jnp.float32),
        pltpu.VMEM((B, F2), jnp.float32),
        pltpu.VMEM((B, F2), jnp.float32),
        pltpu.VMEM((1, 2 * F2), jnp.float32),
        pltpu.SemaphoreType.DMA,
        pltpu.SemaphoreType.DMA,
        pltpu.SemaphoreType.DMA,
    ],
)(_sc_gather_body)


def _act_body(lin_ref, part_ref, g_ref, bt_ref, out_ref):
    part = part_ref[...]
    s = jnp.sum(part[:, :F2], axis=0)
    q = jnp.sum(part[:, F2:], axis=0)
    mu = s * (1.0 / E)
    var = q * (1.0 / E) - mu * mu
    inv = lax.rsqrt(var + 1e-5)
    scale = inv * g_ref[0]
    shift = bt_ref[0] - mu * scale
    y = lin_ref[...] * scale[None, :] + shift[None, :]
    y1 = y[:, :D]
    y2 = y[:, D:]
    sig = jax.nn.sigmoid(y1)
    sp = jnp.maximum(y2, 0.0) + jnp.log1p(jnp.exp(-jnp.abs(y2)))
    out_ref[...] = sig * sp


def _sc_scatter_body(msg_hbm, dst_hbm, zeros_hbm, agg_hbm,
                     di_v, m_v, acc_sh, sem):
    cid = lax.axis_index("c")
    sid = lax.axis_index("s")
    wid = sid * NC + cid
    base = wid * EPW
    rows = pl.ds(sid * NPS, NPS)

    pltpu.sync_copy(zeros_hbm.at[rows], acc_sh.at[rows])
    plsc.subcore_barrier()

    def chunk_body(t, _):
        off = base + t * B
        pltpu.sync_copy(dst_hbm.at[pl.ds(off, B)], di_v)
        pltpu.async_copy(msg_hbm.at[pl.ds(off, B)], m_v, sem).wait()
        pltpu.sync_copy(m_v, acc_sh.at[di_v], add=True)
        return 0

    lax.fori_loop(0, CH, chunk_body, 0)
    plsc.subcore_barrier()
    pltpu.sync_copy(acc_sh.at[rows], agg_hbm.at[cid, rows])


_sc_scatter = functools.partial(
    pl.kernel,
    out_type=jax.ShapeDtypeStruct((NC, NP, D), jnp.float32),
    mesh=_mesh,
    scratch_types=[
        pltpu.VMEM((B,), jnp.int32),
        pltpu.VMEM((B, D), jnp.float32),
        pltpu.VMEM_SHARED((NP, D), jnp.float32),
        pltpu.SemaphoreType.DMA,
    ],
)(_sc_scatter_body)


def _node_body(agg_ref, nf_ref, g_ref, bt_ref, out_ref):
    x = agg_ref[...]
    agg = x[0, :N] + x[1, :N]
    mu = jnp.mean(agg, axis=0)
    var = jnp.mean(agg * agg, axis=0) - mu * mu
    inv = lax.rsqrt(var + 1e-5)
    scale = inv * g_ref[0]
    shift = bt_ref[0] - mu * scale
    out_ref[...] = jax.nn.sigmoid(agg * scale[None, :] + shift[None, :]
                                  + nf_ref[...])


def kernel(node_feats, edge_feats, edge_index,
           mlpt_W, mlpt_b, mlpt_gamma, mlpt_beta,
           gate_W, gate_b, gate_gamma, gate_beta,
           node_gamma, node_beta):
    f32 = jnp.float32
    ws = jnp.concatenate([mlpt_W[:D], gate_W[:D]], axis=1)
    wd = jnp.concatenate([mlpt_W[D:2 * D], gate_W[D:2 * D]], axis=1)
    we = jnp.concatenate([mlpt_W[2 * D:], gate_W[2 * D:]], axis=1)
    g2 = jnp.concatenate([mlpt_gamma, gate_gamma]).reshape(1, F2)
    bt2 = jnp.concatenate([mlpt_beta, gate_beta]).reshape(1, F2)
    src = edge_index[0]
    dst = edge_index[1]

    nb = 2000
    ps, pd = pl.pallas_call(
        _proj_body,
        grid=(N // nb,),
        in_specs=[pl.BlockSpec((nb, D), lambda i: (i, 0)),
                  pl.BlockSpec((D, F2), lambda i: (0, 0)),
                  pl.BlockSpec((D, F2), lambda i: (0, 0))],
        out_specs=[pl.BlockSpec((nb, F2), lambda i: (i, 0)),
                   pl.BlockSpec((nb, F2), lambda i: (i, 0))],
        out_shape=[jax.ShapeDtypeStruct((N, F2), f32),
                   jax.ShapeDtypeStruct((N, F2), f32)],
    )(node_feats, ws, wd)

    eb = 2560
    c = pl.pallas_call(
        _edge_mm_body,
        grid=(E // eb,),
        in_specs=[pl.BlockSpec((eb, D), lambda i: (i, 0)),
                  pl.BlockSpec((D, F2), lambda i: (0, 0))],
        out_specs=pl.BlockSpec((eb, F2), lambda i: (i, 0)),
        out_shape=jax.ShapeDtypeStruct((E, F2), f32),
    )(edge_feats, we)

    lin, part = _sc_gather(ps, pd, src, dst, c)
    part = part.reshape(NW, 2 * F2)

    msg = pl.pallas_call(
        _act_body,
        grid=(E // eb,),
        in_specs=[pl.BlockSpec((eb, F2), lambda i: (i, 0)),
                  pl.BlockSpec((NW, 2 * F2), lambda i: (0, 0)),
                  pl.BlockSpec((1, F2), lambda i: (0, 0)),
                  pl.BlockSpec((1, F2), lambda i: (0, 0))],
        out_specs=pl.BlockSpec((eb, D), lambda i: (i, 0)),
        out_shape=jax.ShapeDtypeStruct((E, D), f32),
    )(lin, part, g2, bt2)

    aggp = _sc_scatter(msg, dst, jnp.zeros((NP, D), f32))

    out_nodes = pl.pallas_call(
        _node_body,
        in_specs=[pl.BlockSpec((NC, NP, D), lambda: (0, 0, 0)),
                  pl.BlockSpec((N, D), lambda: (0, 0)),
                  pl.BlockSpec((1, D), lambda: (0, 0)),
                  pl.BlockSpec((1, D), lambda: (0, 0))],
        out_specs=pl.BlockSpec((N, D), lambda: (0, 0)),
        out_shape=jax.ShapeDtypeStruct((N, D), f32),
    )(aggp, node_feats, node_gamma.reshape(1, D), node_beta.reshape(1, D))

    return out_nodes, edge_feats


# speedup vs baseline: 2.4340x; 1.7376x over previous
import functools

import jax
import jax.numpy as jnp
from jax import lax
from jax.experimental import pallas as pl
from jax.experimental.pallas import tpu as pltpu
from jax.experimental.pallas import tpu_sc as plsc

N = 10000
E = 320000
D = 128
F2 = 2 * D

NC = 2
NS = 16
NW = NC * NS
EPW = E // NW
B = 40
CH = EPW // B
NP = 10240
NPS = NP // NS

_mesh = plsc.VectorSubcoreMesh(core_axis_name="c", subcore_axis_name="s")


def _proj_body(nf_ref, ws_ref, wd_ref, ps_ref, pd_ref):
    x = nf_ref[...]
    ps_ref[...] = jnp.dot(x, ws_ref[...], preferred_element_type=jnp.float32)
    pd_ref[...] = jnp.dot(x, wd_ref[...], preferred_element_type=jnp.float32)


def _edge_mm_body(ef_ref, we_ref, c_ref):
    c_ref[...] = jnp.dot(ef_ref[...], we_ref[...],
                         preferred_element_type=jnp.float32)


def _sc_gather_body(ps_hbm, pd_hbm, src_hbm, dst_hbm, c_hbm,
                    lin_hbm, part_hbm,
                    sia_v, dia_v, a_v, b_v, c_v, w_v, acc_v,
                    gsem0, gsem1, wsem0, wsem1):
    cid = lax.axis_index("c")
    sid = lax.axis_index("s")
    wid = sid * NC + cid
    base = wid * EPW
    gsem = (gsem0, gsem1)
    wsem = (wsem0, wsem1)

    pltpu.sync_copy(src_hbm.at[pl.ds(base, EPW)], sia_v)
    pltpu.sync_copy(dst_hbm.at[pl.ds(base, EPW)], dia_v)

    zero = jnp.zeros((16,), jnp.float32)
    for j in range(2 * F2 // 16):
        acc_v[0, pl.ds(16 * j, 16)] = zero

    def issue(t, k):
        loc = t * B
        pltpu.async_copy(ps_hbm.at[sia_v.at[pl.ds(loc, B)]], a_v.at[k],
                         gsem[k])
        pltpu.async_copy(pd_hbm.at[dia_v.at[pl.ds(loc, B)]], b_v.at[k],
                         gsem[k])
        pltpu.async_copy(c_hbm.at[pl.ds(base + loc, B)], c_v.at[k], gsem[k])

    def drain_gather(k):
        pltpu.make_async_copy(c_hbm.at[pl.ds(base, B)], a_v.at[k],
                              gsem[k]).wait()
        pltpu.make_async_copy(c_hbm.at[pl.ds(base, B)], b_v.at[k],
                              gsem[k]).wait()
        pltpu.make_async_copy(c_hbm.at[pl.ds(base, B)], c_v.at[k],
                              gsem[k]).wait()

    def drain_write(k):
        pltpu.make_async_copy(w_v.at[k], lin_hbm.at[pl.ds(base, B)],
                              wsem[k]).wait()

    issue(0, 0)

    def outer(g, _):
        for k in range(2):
            t = 2 * g + k
            kn = 1 - k

            @pl.when(t + 1 < CH)
            def _():
                issue(t + 1, kn)

            drain_gather(k)

            @pl.when(t >= 2)
            def _():
                drain_write(k)

            for j in range(F2 // 16):
                cs = pl.ds(16 * j, 16)

                def row_body(r, carry):
                    s, q = carry
                    v = a_v[k, r, cs] + b_v[k, r, cs] + c_v[k, r, cs]
                    w_v[k, r, cs] = v
                    return s + v, q + v * v

                s, q = lax.fori_loop(0, B, row_body, (zero, zero))
                acc_v[0, cs] = acc_v[0, cs] + s
                qs = pl.ds(F2 + 16 * j, 16)
                acc_v[0, qs] = acc_v[0, qs] + q

            pltpu.async_copy(w_v.at[k], lin_hbm.at[pl.ds(base + t * B, B)],
                             wsem[k])
        return 0

    lax.fori_loop(0, CH // 2, outer, 0)
    drain_write(0)
    drain_write(1)
    pltpu.sync_copy(acc_v, part_hbm.at[wid])


_sc_gather = functools.partial(
    pl.kernel,
    out_type=[jax.ShapeDtypeStruct((E, F2), jnp.float32),
              jax.ShapeDtypeStruct((NW, 1, 2 * F2), jnp.float32)],
    mesh=_mesh,
    scratch_types=[
        pltpu.VMEM((EPW,), jnp.int32),
        pltpu.VMEM((EPW,), jnp.int32),
        pltpu.VMEM((2, B, F2), jnp.float32),
        pltpu.VMEM((2, B, F2), jnp.float32),
        pltpu.VMEM((2, B, F2), jnp.float32),
        pltpu.VMEM((2, B, F2), jnp.float32),
        pltpu.VMEM((1, 2 * F2), jnp.float32),
        pltpu.SemaphoreType.DMA,
        pltpu.SemaphoreType.DMA,
        pltpu.SemaphoreType.DMA,
        pltpu.SemaphoreType.DMA,
    ],
)(_sc_gather_body)


def _act_body(lin_ref, part_ref, g_ref, bt_ref, out_ref):
    part = part_ref[...]
    s = jnp.sum(part[:, :F2], axis=0)
    q = jnp.sum(part[:, F2:], axis=0)
    mu = s * (1.0 / E)
    var = q * (1.0 / E) - mu * mu
    inv = lax.rsqrt(var + 1e-5)
    scale = inv * g_ref[0]
    shift = bt_ref[0] - mu * scale
    y = lin_ref[...] * scale[None, :] + shift[None, :]
    y1 = y[:, :D]
    y2 = y[:, D:]
    sig = jax.nn.sigmoid(y1)
    sp = jnp.maximum(y2, 0.0) + jnp.log1p(jnp.exp(-jnp.abs(y2)))
    out_ref[...] = sig * sp


def _sc_scatter_body(msg_hbm, dst_hbm, zeros_hbm, agg_hbm,
                     di0_v, di1_v, m_v, acc_sh, isem0, isem1, msem0, msem1):
    cid = lax.axis_index("c")
    sid = lax.axis_index("s")
    wid = sid * NC + cid
    base = wid * EPW
    rows = pl.ds(sid * NPS, NPS)
    di = (di0_v, di1_v)
    isem = (isem0, isem1)
    msem = (msem0, msem1)

    pltpu.sync_copy(zeros_hbm.at[rows], acc_sh.at[rows])

    def issue(t, k):
        off = base + t * B
        pltpu.async_copy(dst_hbm.at[pl.ds(off, B)], di[k], isem[k])
        pltpu.async_copy(msg_hbm.at[pl.ds(off, B)], m_v.at[k], msem[k])

    def drain(k):
        pltpu.make_async_copy(dst_hbm.at[pl.ds(base, B)], di[k],
                              isem[k]).wait()
        pltpu.make_async_copy(msg_hbm.at[pl.ds(base, B)], m_v.at[k],
                              msem[k]).wait()

    plsc.subcore_barrier()
    issue(0, 0)

    def outer(g, _):
        for k in range(2):
            t = 2 * g + k
            kn = 1 - k

            @pl.when(t + 1 < CH)
            def _():
                issue(t + 1, kn)

            drain(k)
            pltpu.sync_copy(m_v.at[k], acc_sh.at[di[k]], add=True)
        return 0

    lax.fori_loop(0, CH // 2, outer, 0)
    plsc.subcore_barrier()
    pltpu.sync_copy(acc_sh.at[rows], agg_hbm.at[cid, rows])


_sc_scatter = functools.partial(
    pl.kernel,
    out_type=jax.ShapeDtypeStruct((NC, NP, D), jnp.float32),
    mesh=_mesh,
    scratch_types=[
        pltpu.VMEM((B,), jnp.int32),
        pltpu.VMEM((B,), jnp.int32),
        pltpu.VMEM((2, B, D), jnp.float32),
        pltpu.VMEM_SHARED((NP, D), jnp.float32),
        pltpu.SemaphoreType.DMA,
        pltpu.SemaphoreType.DMA,
        pltpu.SemaphoreType.DMA,
        pltpu.SemaphoreType.DMA,
    ],
)(_sc_scatter_body)


def _node_body(agg_ref, nf_ref, g_ref, bt_ref, out_ref):
    x = agg_ref[...]
    agg = x[0, :N] + x[1, :N]
    mu = jnp.mean(agg, axis=0)
    var = jnp.mean(agg * agg, axis=0) - mu * mu
    inv = lax.rsqrt(var + 1e-5)
    scale = inv * g_ref[0]
    shift = bt_ref[0] - mu * scale
    out_ref[...] = jax.nn.sigmoid(agg * scale[None, :] + shift[None, :]
                                  + nf_ref[...])


def kernel(node_feats, edge_feats, edge_index,
           mlpt_W, mlpt_b, mlpt_gamma, mlpt_beta,
           gate_W, gate_b, gate_gamma, gate_beta,
           node_gamma, node_beta):
    f32 = jnp.float32
    ws = jnp.concatenate([mlpt_W[:D], gate_W[:D]], axis=1)
    wd = jnp.concatenate([mlpt_W[D:2 * D], gate_W[D:2 * D]], axis=1)
    we = jnp.concatenate([mlpt_W[2 * D:], gate_W[2 * D:]], axis=1)
    g2 = jnp.concatenate([mlpt_gamma, gate_gamma]).reshape(1, F2)
    bt2 = jnp.concatenate([mlpt_beta, gate_beta]).reshape(1, F2)
    src = edge_index[0]
    dst = edge_index[1]

    nb = 2000
    ps, pd = pl.pallas_call(
        _proj_body,
        grid=(N // nb,),
        in_specs=[pl.BlockSpec((nb, D), lambda i: (i, 0)),
                  pl.BlockSpec((D, F2), lambda i: (0, 0)),
                  pl.BlockSpec((D, F2), lambda i: (0, 0))],
        out_specs=[pl.BlockSpec((nb, F2), lambda i: (i, 0)),
                   pl.BlockSpec((nb, F2), lambda i: (i, 0))],
        out_shape=[jax.ShapeDtypeStruct((N, F2), f32),
                   jax.ShapeDtypeStruct((N, F2), f32)],
    )(node_feats, ws, wd)

    eb = 2560
    c = pl.pallas_call(
        _edge_mm_body,
        grid=(E // eb,),
        in_specs=[pl.BlockSpec((eb, D), lambda i: (i, 0)),
                  pl.BlockSpec((D, F2), lambda i: (0, 0))],
        out_specs=pl.BlockSpec((eb, F2), lambda i: (i, 0)),
        out_shape=jax.ShapeDtypeStruct((E, F2), f32),
    )(edge_feats, we)

    lin, part = _sc_gather(ps, pd, src, dst, c)
    part = part.reshape(NW, 2 * F2)

    msg = pl.pallas_call(
        _act_body,
        grid=(E // eb,),
        in_specs=[pl.BlockSpec((eb, F2), lambda i: (i, 0)),
                  pl.BlockSpec((NW, 2 * F2), lambda i: (0, 0)),
                  pl.BlockSpec((1, F2), lambda i: (0, 0)),
                  pl.BlockSpec((1, F2), lambda i: (0, 0))],
        out_specs=pl.BlockSpec((eb, D), lambda i: (i, 0)),
        out_shape=jax.ShapeDtypeStruct((E, D), f32),
    )(lin, part, g2, bt2)

    aggp = _sc_scatter(msg, dst, jnp.zeros((NP, D), f32))

    out_nodes = pl.pallas_call(
        _node_body,
        in_specs=[pl.BlockSpec((NC, NP, D), lambda: (0, 0, 0)),
                  pl.BlockSpec((N, D), lambda: (0, 0)),
                  pl.BlockSpec((1, D), lambda: (0, 0)),
                  pl.BlockSpec((1, D), lambda: (0, 0))],
        out_specs=pl.BlockSpec((N, D), lambda: (0, 0)),
        out_shape=jax.ShapeDtypeStruct((N, D), f32),
    )(aggp, node_feats, node_gamma.reshape(1, D), node_beta.reshape(1, D))

    return out_nodes, edge_feats


# speedup vs baseline: 2.6204x; 1.0766x over previous
import functools

import jax
import jax.numpy as jnp
import numpy as np
from jax import lax
from jax.experimental import pallas as pl
from jax.experimental.pallas import tpu as pltpu
from jax.experimental.pallas import tpu_sc as plsc

N = 10000
E = 320000
D = 128
F2 = 2 * D

NC = 2
NS = 16
NW = NC * NS
EPW = E // NW
B = 40
CH = EPW // B
NP = 10240
NPS = NP // NS

_mesh = plsc.VectorSubcoreMesh(core_axis_name="c", subcore_axis_name="s")

_PERM = np.concatenate(
    [np.arange(16) + 32 * m for m in range(F2 // 32)]
    + [np.arange(16) + 32 * m + 16 for m in range(F2 // 32)]).astype(np.int32)


def _pack_words(x):
    hi = x[:, :F2 // 2].astype(jnp.bfloat16)
    lo = x[:, F2 // 2:].astype(jnp.bfloat16)
    hu = lax.convert_element_type(lax.bitcast_convert_type(hi, jnp.uint16),
                                  jnp.uint32)
    lu = lax.convert_element_type(lax.bitcast_convert_type(lo, jnp.uint16),
                                  jnp.uint32)
    return lax.bitcast_convert_type((hu << 16) | lu, jnp.float32)


def _proj_body(nf_ref, ws_ref, wd_ref, ps_ref, pd_ref):
    x = nf_ref[...]
    ps_ref[...] = _pack_words(
        jnp.dot(x, ws_ref[...], preferred_element_type=jnp.float32))
    pd_ref[...] = _pack_words(
        jnp.dot(x, wd_ref[...], preferred_element_type=jnp.float32))


def _edge_mm_body(ef_ref, we_ref, c_ref):
    c_ref[...] = _pack_words(
        jnp.dot(ef_ref[...], we_ref[...], preferred_element_type=jnp.float32))


def _sc_gather_body(ps_hbm, pd_hbm, src_hbm, dst_hbm, c_hbm,
                    lin_hbm, part_hbm,
                    sia_v, dia_v, a_v, b_v, c_v, w_v, acc_v,
                    gsem0, gsem1, wsem0, wsem1):
    cid = lax.axis_index("c")
    sid = lax.axis_index("s")
    wid = sid * NC + cid
    base = wid * EPW
    gsem = (gsem0, gsem1)
    wsem = (wsem0, wsem1)

    pltpu.sync_copy(src_hbm.at[pl.ds(base, EPW)], sia_v)
    pltpu.sync_copy(dst_hbm.at[pl.ds(base, EPW)], dia_v)

    zero = jnp.zeros((16,), jnp.float32)
    for j in range(2 * F2 // 16):
        acc_v[0, pl.ds(16 * j, 16)] = zero

    def issue(t, k):
        loc = t * B
        pltpu.async_copy(ps_hbm.at[sia_v.at[pl.ds(loc, B)]], a_v.at[k],
                         gsem[k])
        pltpu.async_copy(pd_hbm.at[dia_v.at[pl.ds(loc, B)]], b_v.at[k],
                         gsem[k])
        pltpu.async_copy(c_hbm.at[pl.ds(base + loc, B)], c_v.at[k], gsem[k])

    def drain_gather(k):
        pltpu.make_async_copy(c_hbm.at[pl.ds(base, B)], a_v.at[k],
                              gsem[k]).wait()
        pltpu.make_async_copy(c_hbm.at[pl.ds(base, B)], b_v.at[k],
                              gsem[k]).wait()
        pltpu.make_async_copy(c_hbm.at[pl.ds(base, B)], c_v.at[k],
                              gsem[k]).wait()

    def drain_write(k):
        pltpu.make_async_copy(w_v.at[k], lin_hbm.at[pl.ds(base, B)],
                              wsem[k]).wait()

    issue(0, 0)

    def outer(g, _):
        for k in range(2):
            t = 2 * g + k
            kn = 1 - k

            @pl.when(t + 1 < CH)
            def _():
                issue(t + 1, kn)

            drain_gather(k)

            @pl.when(t >= 2)
            def _():
                drain_write(k)

            for j in range(F2 // 32):
                wsl = pl.ds(16 * j, 16)

                def row_body(r, carry):
                    s0, q0, s1, q1 = carry
                    ua = plsc.bitcast(a_v[k, r, wsl], jnp.int32)
                    ub = plsc.bitcast(b_v[k, r, wsl], jnp.int32)
                    uc = plsc.bitcast(c_v[k, r, wsl], jnp.int32)
                    hmask = jnp.int32(-65536)
                    a0 = plsc.bitcast(ua & hmask, jnp.float32)
                    b0 = plsc.bitcast(ub & hmask, jnp.float32)
                    c0 = plsc.bitcast(uc & hmask, jnp.float32)
                    a1 = plsc.bitcast(ua << 16, jnp.float32)
                    b1 = plsc.bitcast(ub << 16, jnp.float32)
                    c1 = plsc.bitcast(uc << 16, jnp.float32)
                    v0 = a0 + b0 + c0
                    v1 = a1 + b1 + c1
                    w_v[k, r, pl.ds(32 * j, 16)] = v0
                    w_v[k, r, pl.ds(32 * j + 16, 16)] = v1
                    return s0 + v0, q0 + v0 * v0, s1 + v1, q1 + v1 * v1

                s0, q0, s1, q1 = lax.fori_loop(0, B, row_body,
                                               (zero, zero, zero, zero))
                cs0 = pl.ds(32 * j, 16)
                cs1 = pl.ds(32 * j + 16, 16)
                qs0 = pl.ds(F2 + 32 * j, 16)
                qs1 = pl.ds(F2 + 32 * j + 16, 16)
                acc_v[0, cs0] = acc_v[0, cs0] + s0
                acc_v[0, cs1] = acc_v[0, cs1] + s1
                acc_v[0, qs0] = acc_v[0, qs0] + q0
                acc_v[0, qs1] = acc_v[0, qs1] + q1

            pltpu.async_copy(w_v.at[k], lin_hbm.at[pl.ds(base + t * B, B)],
                             wsem[k])
        return 0

    lax.fori_loop(0, CH // 2, outer, 0)
    drain_write(0)
    drain_write(1)
    pltpu.sync_copy(acc_v, part_hbm.at[wid])


_sc_gather = functools.partial(
    pl.kernel,
    out_type=[jax.ShapeDtypeStruct((E, F2), jnp.float32),
              jax.ShapeDtypeStruct((NW, 1, 2 * F2), jnp.float32)],
    mesh=_mesh,
    scratch_types=[
        pltpu.VMEM((EPW,), jnp.int32),
        pltpu.VMEM((EPW,), jnp.int32),
        pltpu.VMEM((2, B, F2 // 2), jnp.float32),
        pltpu.VMEM((2, B, F2 // 2), jnp.float32),
        pltpu.VMEM((2, B, F2 // 2), jnp.float32),
        pltpu.VMEM((2, B, F2), jnp.float32),
        pltpu.VMEM((1, 2 * F2), jnp.float32),
        pltpu.SemaphoreType.DMA,
        pltpu.SemaphoreType.DMA,
        pltpu.SemaphoreType.DMA,
        pltpu.SemaphoreType.DMA,
    ],
    compiler_params=pltpu.CompilerParams(needs_layout_passes=False),
)(_sc_gather_body)


def _act_body(lin_ref, part_ref, g_ref, bt_ref, out_ref):
    part = part_ref[...]
    s = jnp.sum(part[:, :F2], axis=0)
    q = jnp.sum(part[:, F2:], axis=0)
    mu = s * (1.0 / E)
    var = q * (1.0 / E) - mu * mu
    inv = lax.rsqrt(var + 1e-5)
    scale = inv * g_ref[0]
    shift = bt_ref[0] - mu * scale
    y = lin_ref[...] * scale[None, :] + shift[None, :]
    y1 = y[:, :D]
    y2 = y[:, D:]
    sig = jax.nn.sigmoid(y1)
    sp = jnp.maximum(y2, 0.0) + jnp.log1p(jnp.exp(-jnp.abs(y2)))
    out_ref[...] = sig * sp


def _sc_scatter_body(msg_hbm, dst_hbm, zeros_hbm, agg_hbm,
                     di0_v, di1_v, m_v, acc_sh, isem0, isem1, msem0, msem1):
    cid = lax.axis_index("c")
    sid = lax.axis_index("s")
    wid = sid * NC + cid
    base = wid * EPW
    rows = pl.ds(sid * NPS, NPS)
    di = (di0_v, di1_v)
    isem = (isem0, isem1)
    msem = (msem0, msem1)

    pltpu.sync_copy(zeros_hbm.at[rows], acc_sh.at[rows])

    def issue(t, k):
        off = base + t * B
        pltpu.async_copy(dst_hbm.at[pl.ds(off, B)], di[k], isem[k])
        pltpu.async_copy(msg_hbm.at[pl.ds(off, B)], m_v.at[k], msem[k])

    def drain(k):
        pltpu.make_async_copy(dst_hbm.at[pl.ds(base, B)], di[k],
                              isem[k]).wait()
        pltpu.make_async_copy(msg_hbm.at[pl.ds(base, B)], m_v.at[k],
                              msem[k]).wait()

    plsc.subcore_barrier()
    issue(0, 0)

    def outer(g, _):
        for k in range(2):
            t = 2 * g + k
            kn = 1 - k

            @pl.when(t + 1 < CH)
            def _():
                issue(t + 1, kn)

            drain(k)
            pltpu.sync_copy(m_v.at[k], acc_sh.at[di[k]], add=True)
        return 0

    lax.fori_loop(0, CH // 2, outer, 0)
    plsc.subcore_barrier()
    pltpu.sync_copy(acc_sh.at[rows], agg_hbm.at[cid, rows])


_sc_scatter = functools.partial(
    pl.kernel,
    out_type=jax.ShapeDtypeStruct((NC, NP, D), jnp.float32),
    mesh=_mesh,
    scratch_types=[
        pltpu.VMEM((B,), jnp.int32),
        pltpu.VMEM((B,), jnp.int32),
        pltpu.VMEM((2, B, D), jnp.float32),
        pltpu.VMEM_SHARED((NP, D), jnp.float32),
        pltpu.SemaphoreType.DMA,
        pltpu.SemaphoreType.DMA,
        pltpu.SemaphoreType.DMA,
        pltpu.SemaphoreType.DMA,
    ],
)(_sc_scatter_body)


def _node_body(agg_ref, nf_ref, g_ref, bt_ref, out_ref):
    x = agg_ref[...]
    agg = x[0, :N] + x[1, :N]
    mu = jnp.mean(agg, axis=0)
    var = jnp.mean(agg * agg, axis=0) - mu * mu
    inv = lax.rsqrt(var + 1e-5)
    scale = inv * g_ref[0]
    shift = bt_ref[0] - mu * scale
    out_ref[...] = jax.nn.sigmoid(agg * scale[None, :] + shift[None, :]
                                  + nf_ref[...])


def kernel(node_feats, edge_feats, edge_index,
           mlpt_W, mlpt_b, mlpt_gamma, mlpt_beta,
           gate_W, gate_b, gate_gamma, gate_beta,
           node_gamma, node_beta):
    f32 = jnp.float32
    ws = jnp.concatenate([mlpt_W[:D], gate_W[:D]], axis=1)[:, _PERM]
    wd = jnp.concatenate([mlpt_W[D:2 * D], gate_W[D:2 * D]], axis=1)[:, _PERM]
    we = jnp.concatenate([mlpt_W[2 * D:], gate_W[2 * D:]], axis=1)[:, _PERM]
    g2 = jnp.concatenate([mlpt_gamma, gate_gamma]).reshape(1, F2)
    bt2 = jnp.concatenate([mlpt_beta, gate_beta]).reshape(1, F2)
    src = edge_index[0]
    dst = edge_index[1]

    nb = 2000
    ps, pd = pl.pallas_call(
        _proj_body,
        grid=(N // nb,),
        in_specs=[pl.BlockSpec((nb, D), lambda i: (i, 0)),
                  pl.BlockSpec((D, F2), lambda i: (0, 0)),
                  pl.BlockSpec((D, F2), lambda i: (0, 0))],
        out_specs=[pl.BlockSpec((nb, F2 // 2), lambda i: (i, 0)),
                   pl.BlockSpec((nb, F2 // 2), lambda i: (i, 0))],
        out_shape=[jax.ShapeDtypeStruct((N, F2 // 2), f32),
                   jax.ShapeDtypeStruct((N, F2 // 2), f32)],
    )(node_feats, ws, wd)

    eb = 2560
    c = pl.pallas_call(
        _edge_mm_body,
        grid=(E // eb,),
        in_specs=[pl.BlockSpec((eb, D), lambda i: (i, 0)),
                  pl.BlockSpec((D, F2), lambda i: (0, 0))],
        out_specs=pl.BlockSpec((eb, F2 // 2), lambda i: (i, 0)),
        out_shape=jax.ShapeDtypeStruct((E, F2 // 2), f32),
    )(edge_feats, we)

    lin, part = _sc_gather(ps, pd, src, dst, c)
    part = part.reshape(NW, 2 * F2)

    msg = pl.pallas_call(
        _act_body,
        grid=(E // eb,),
        in_specs=[pl.BlockSpec((eb, F2), lambda i: (i, 0)),
                  pl.BlockSpec((NW, 2 * F2), lambda i: (0, 0)),
                  pl.BlockSpec((1, F2), lambda i: (0, 0)),
                  pl.BlockSpec((1, F2), lambda i: (0, 0))],
        out_specs=pl.BlockSpec((eb, D), lambda i: (i, 0)),
        out_shape=jax.ShapeDtypeStruct((E, D), f32),
    )(lin, part, g2, bt2)

    aggp = _sc_scatter(msg, dst, jnp.zeros((NP, D), f32))

    out_nodes = pl.pallas_call(
        _node_body,
        in_specs=[pl.BlockSpec((NC, NP, D), lambda: (0, 0, 0)),
                  pl.BlockSpec((N, D), lambda: (0, 0)),
                  pl.BlockSpec((1, D), lambda: (0, 0)),
                  pl.BlockSpec((1, D), lambda: (0, 0))],
        out_specs=pl.BlockSpec((N, D), lambda: (0, 0)),
        out_shape=jax.ShapeDtypeStruct((N, D), f32),
    )(aggp, node_feats, node_gamma.reshape(1, D), node_beta.reshape(1, D))

    return out_nodes, edge_feats
